# baseline (speedup 1.0000x reference)
import functools

import jax
import jax.numpy as jnp
from jax import lax
from jax.experimental import pallas as pl
from jax.experimental.pallas import tpu as pltpu
from jax.experimental.pallas import tpu_sc as plsc

N = 10000
E = 320000
D = 128
H = 128
OUT = 64
G = 64

NC = 2
NS = 16
NPAD = 10240
CHUNK = 128
BLK = 2
EPAD = 327680

_vector_mesh = plsc.VectorSubcoreMesh(core_axis_name="c", subcore_axis_name="s")


def _sc_degree(dst2d, ones_c, zeros_n):
    rows = dst2d.shape[0]
    rows_per_w = rows // (NC * NS)
    n_blocks = rows_per_w // BLK
    stripe = NPAD // NS

    @functools.partial(
        pl.kernel,
        out_type=jax.ShapeDtypeStruct((NC, NPAD), jnp.float32),
        mesh=_vector_mesh,
        scratch_types=[
            pltpu.VMEM((rows // (NC * NS), CHUNK), jnp.int32),
            pltpu.VMEM((CHUNK,), jnp.float32),
            pltpu.VMEM_SHARED((NPAD,), jnp.float32),
            pltpu.SemaphoreType.DMA,
        ],
    )
    def k(dst_hbm, ones_hbm, zeros_hbm, out_hbm, idx_v, ones_v, acc, ssem):
        c = lax.axis_index("c")
        s = lax.axis_index("s")
        w = c * NS + s
        pltpu.sync_copy(zeros_hbm.at[pl.ds(s * stripe, stripe)],
                        acc.at[pl.ds(s * stripe, stripe)])
        pltpu.sync_copy(ones_hbm, ones_v)
        pltpu.sync_copy(dst_hbm.at[pl.ds(w * rows_per_w, rows_per_w)], idx_v)
        plsc.subcore_barrier()
        scatters = [
            pltpu.async_copy(ones_v, acc.at[idx_v.at[j]], ssem, add=True)
            for j in range(rows_per_w)
        ]
        for cp in scatters:
            cp.wait()
        plsc.subcore_barrier()
        pltpu.sync_copy(acc.at[pl.ds(s * stripe, stripe)],
                        out_hbm.at[c].at[pl.ds(s * stripe, stripe)])

    return k(dst2d, ones_c, zeros_n)


def _sc_scatter(hp, src2d, dst2d, zeros2d):
    rows = src2d.shape[0]
    rows_per_tile = rows // (NC * NS)
    SB = 16
    n_sb = rows_per_tile // SB
    stripe = NPAD // NS

    @functools.partial(
        pl.kernel,
        out_type=jax.ShapeDtypeStruct((NC, NPAD, H), jnp.float32),
        mesh=_vector_mesh,
        scratch_types=[
            pltpu.VMEM((2, SB, CHUNK), jnp.int32),
            pltpu.VMEM((2, SB, CHUNK), jnp.int32),
            pltpu.VMEM((CHUNK, H), jnp.float32),
            pltpu.VMEM((CHUNK, H), jnp.float32),
            pltpu.VMEM_SHARED((NPAD, H), jnp.float32),
            pltpu.SemaphoreType.DMA,
            pltpu.SemaphoreType.DMA,
            pltpu.SemaphoreType.DMA,
            pltpu.SemaphoreType.DMA,
            pltpu.SemaphoreType.DMA,
        ],
    )
    def k(hp_hbm, src_hbm, dst_hbm, zeros_hbm, out_hbm, src_v, dst_v,
          rows_a, rows_b, acc, gsem, hsem, isem0, isem1, initsem):
        c = lax.axis_index("c")
        s = lax.axis_index("s")
        base = (c * NS + s) * rows_per_tile

        @pl.when(c == 0)
        def _():
            pltpu.async_copy(hp_hbm.at[pl.ds(s * stripe, stripe)],
                             acc.at[pl.ds(s * stripe, stripe)], initsem)

        @pl.when(c != 0)
        def _():
            pltpu.async_copy(zeros_hbm.at[pl.ds(s * stripe, stripe)],
                             acc.at[pl.ds(s * stripe, stripe)], initsem)

        pltpu.async_copy(src_hbm.at[pl.ds(base, SB)], src_v.at[0], isem0)
        pltpu.async_copy(dst_hbm.at[pl.ds(base, SB)], dst_v.at[0], isem0)
        pltpu.make_async_copy(src_hbm.at[pl.ds(base, SB)], src_v.at[0],
                              isem0).wait()
        pltpu.make_async_copy(dst_hbm.at[pl.ds(base, SB)], dst_v.at[0],
                              isem0).wait()
        def issue_gather(idx_row, buf, sem):
            pltpu.async_copy(hp_hbm.at[idx_row.at[pl.ds(0, 64)]],
                             buf.at[pl.ds(0, 64)], sem)
            pltpu.async_copy(hp_hbm.at[idx_row.at[pl.ds(64, 64)]],
                             buf.at[pl.ds(64, 64)], sem)

        def wait_gather(idx_row, buf, sem):
            pltpu.make_async_copy(hp_hbm.at[idx_row.at[pl.ds(0, 64)]],
                                  buf.at[pl.ds(0, 64)], sem).wait()
            pltpu.make_async_copy(hp_hbm.at[idx_row.at[pl.ds(64, 64)]],
                                  buf.at[pl.ds(64, 64)], sem).wait()

        issue_gather(src_v.at[0].at[0], rows_a, gsem)
        issue_gather(src_v.at[0].at[1], rows_b, hsem)
        pltpu.make_async_copy(hp_hbm.at[pl.ds(s * stripe, stripe)],
                              acc.at[pl.ds(s * stripe, stripe)],
                              initsem).wait()
        plsc.subcore_barrier()

        @pl.loop(0, n_sb)
        def _(sb):
            p = sb % 2
            r_next = base + (sb + 1) * SB

            @pl.when(sb < n_sb - 1)
            def _():
                pltpu.async_copy(src_hbm.at[pl.ds(r_next, SB)],
                                 src_v.at[1 - p], isem1)
                pltpu.async_copy(dst_hbm.at[pl.ds(r_next, SB)],
                                 dst_v.at[1 - p], isem1)

            sidx = src_v.at[p]
            didx = dst_v.at[p]

            @pl.loop(0, SB // 2)
            def _(i):
                ca = 2 * i
                wait_gather(sidx.at[ca], rows_a, gsem)
                pltpu.sync_copy(rows_a, acc.at[didx.at[ca]], add=True)

                @pl.when(i < SB // 2 - 1)
                def _():
                    issue_gather(sidx.at[ca + 2], rows_a, gsem)

                wait_gather(sidx.at[ca + 1], rows_b, hsem)
                pltpu.sync_copy(rows_b, acc.at[didx.at[ca + 1]], add=True)

                @pl.when(i < SB // 2 - 1)
                def _():
                    issue_gather(sidx.at[ca + 3], rows_b, hsem)

            @pl.when(sb < n_sb - 1)
            def _():
                pltpu.make_async_copy(src_hbm.at[pl.ds(r_next, SB)],
                                      src_v.at[1 - p], isem1).wait()
                pltpu.make_async_copy(dst_hbm.at[pl.ds(r_next, SB)],
                                      dst_v.at[1 - p], isem1).wait()
                issue_gather(src_v.at[1 - p].at[0], rows_a, gsem)
                issue_gather(src_v.at[1 - p].at[1], rows_b, hsem)

        plsc.subcore_barrier()
        pltpu.sync_copy(acc.at[pl.ds(s * stripe, stripe)],
                        out_hbm.at[c].at[pl.ds(s * stripe, stripe)])

    return k(hp, src2d, dst2d, zeros2d)


def _dinv_col(hist_ref):
    return lax.rsqrt(1.0 + hist_ref[0] + hist_ref[1])


def _pad_rows(h):
    return jnp.concatenate(
        [h, jnp.zeros((NPAD - N, h.shape[1]), jnp.float32)], axis=0)


def _tc_mm1(x, W1):
    def body(x_ref, w_ref, out_ref):
        out_ref[...] = jnp.dot(x_ref[...], w_ref[...],
                               preferred_element_type=jnp.float32)

    return pl.pallas_call(
        body,
        out_shape=jax.ShapeDtypeStruct((N, H), jnp.float32),
    )(x, W1)


def _tc_scale1(h, hist3):
    def body(h_ref, hist_ref, out_ref):
        dinv = _dinv_col(hist_ref)
        out_ref[...] = _pad_rows(h_ref[...] * dinv[:N])

    return pl.pallas_call(
        body,
        out_shape=jax.ShapeDtypeStruct((NPAD, H), jnp.float32),
    )(h, hist3)


def _tc_layer2(agg1, hist3, b1, W2):
    def body(agg_ref, hist_ref, b_ref, w_ref, out_ref):
        dinv = _dinv_col(hist_ref)
        aggc = agg_ref[0, :N] + agg_ref[1, :N]
        out1 = jax.nn.relu(aggc * dinv[:N] + b_ref[...])
        h2 = jnp.dot(out1, w_ref[...], preferred_element_type=jnp.float32)
        out_ref[...] = _pad_rows(h2 * dinv[:N])

    return pl.pallas_call(
        body,
        out_shape=jax.ShapeDtypeStruct((NPAD, H), jnp.float32),
    )(agg1, hist3, b1, W2)


def _tc_pool(agg2, hist3, b2, batch2d, W3, b3):
    def body(agg_ref, hist_ref, b2_ref, batch_ref, w3_ref, b3_ref, out_ref):
        dinv = _dinv_col(hist_ref)
        aggc = agg_ref[0, :N] + agg_ref[1, :N]
        out2 = jax.nn.relu(aggc * dinv[:N] + b2_ref[...])
        gids = lax.broadcasted_iota(jnp.int32, (G, 1), 0)
        P = (batch_ref[...] == gids).astype(jnp.float32)
        counts = jnp.sum(P, axis=1, keepdims=True)
        pooled = jnp.dot(P, out2, preferred_element_type=jnp.float32)
        pooled = pooled / jnp.maximum(counts, 1.0)
        out_ref[...] = (
            jnp.dot(pooled, w3_ref[...], preferred_element_type=jnp.float32)
            + b3_ref[...])

    return pl.pallas_call(
        body,
        out_shape=jax.ShapeDtypeStruct((G, OUT), jnp.float32),
    )(agg2, hist3, b2, batch2d, W3, b3)


def kernel(x, edge_index, batch, W1, b1, W2, b2, W3, b3):
    src = edge_index[0]
    dst = edge_index[1]
    pad_ids = N + (jnp.arange(EPAD - E, dtype=jnp.int32) % (NPAD - N))
    src2d = jnp.concatenate([src, pad_ids]).reshape(-1, CHUNK)
    dst2d = jnp.concatenate([dst, pad_ids]).reshape(-1, CHUNK)
    ones_c = jnp.ones((CHUNK,), jnp.float32)
    zeros_n = jnp.zeros((NPAD,), jnp.float32)
    zeros2d = jnp.zeros((NPAD, H), jnp.float32)

    hist = _sc_degree(dst2d, ones_c, zeros_n)
    h1 = _tc_mm1(x, W1)
    hist3 = hist.reshape(NC, NPAD, 1)
    hp1 = _tc_scale1(h1, hist3)
    agg1 = _sc_scatter(hp1, src2d, dst2d, zeros2d)
    hp2 = _tc_layer2(agg1, hist3, b1.reshape(1, H), W2)
    agg2 = _sc_scatter(hp2, src2d, dst2d, zeros2d)
    return _tc_pool(agg2, hist3, b2.reshape(1, H), batch.reshape(1, N),
                    W3, b3.reshape(1, OUT))


# speedup vs baseline: 1.0068x; 1.0068x over previous
import functools

import jax
import jax.numpy as jnp
from jax import lax
from jax.experimental import pallas as pl
from jax.experimental.pallas import tpu as pltpu
from jax.experimental.pallas import tpu_sc as plsc

N = 10000
E = 320000
D = 128
H = 128
OUT = 64
G = 64

NC = 2
NS = 16
NPAD = 10240
CHUNK = 128
BLK = 2
EPAD = 327680

_vector_mesh = plsc.VectorSubcoreMesh(core_axis_name="c", subcore_axis_name="s")


def _sc_degree(dst2d, ones_c, zeros_n):
    rows = dst2d.shape[0]
    rows_per_w = rows // (NC * NS)
    n_blocks = rows_per_w // BLK
    stripe = NPAD // NS

    @functools.partial(
        pl.kernel,
        out_type=jax.ShapeDtypeStruct((NC, NPAD), jnp.float32),
        mesh=_vector_mesh,
        scratch_types=[
            pltpu.VMEM((rows // (NC * NS), CHUNK), jnp.int32),
            pltpu.VMEM((CHUNK,), jnp.float32),
            pltpu.VMEM_SHARED((NPAD,), jnp.float32),
            pltpu.SemaphoreType.DMA,
        ],
    )
    def k(dst_hbm, ones_hbm, zeros_hbm, out_hbm, idx_v, ones_v, acc, ssem):
        c = lax.axis_index("c")
        s = lax.axis_index("s")
        w = c * NS + s
        pltpu.sync_copy(zeros_hbm.at[pl.ds(s * stripe, stripe)],
                        acc.at[pl.ds(s * stripe, stripe)])
        pltpu.sync_copy(ones_hbm, ones_v)
        pltpu.sync_copy(dst_hbm.at[pl.ds(w * rows_per_w, rows_per_w)], idx_v)
        plsc.subcore_barrier()
        scatters = [
            pltpu.async_copy(ones_v, acc.at[idx_v.at[j]], ssem, add=True)
            for j in range(rows_per_w)
        ]
        for cp in scatters:
            cp.wait()
        plsc.subcore_barrier()
        pltpu.sync_copy(acc.at[pl.ds(s * stripe, stripe)],
                        out_hbm.at[c].at[pl.ds(s * stripe, stripe)])

    return k(dst2d, ones_c, zeros_n)


def _sc_scatter(hp, src2d, dst2d, zeros2d):
    rows = src2d.shape[0]
    rows_per_tile = rows // (NC * NS)
    SB = 16
    n_sb = rows_per_tile // SB
    stripe = NPAD // NS

    @functools.partial(
        pl.kernel,
        out_type=jax.ShapeDtypeStruct((NC, NPAD, H), jnp.float32),
        mesh=_vector_mesh,
        scratch_types=[
            pltpu.VMEM((2, SB, CHUNK), jnp.int32),
            pltpu.VMEM((2, SB, CHUNK), jnp.int32),
            pltpu.VMEM((CHUNK, H), jnp.float32),
            pltpu.VMEM((CHUNK, H), jnp.float32),
            pltpu.VMEM_SHARED((NPAD, H), jnp.float32),
            pltpu.SemaphoreType.DMA,
            pltpu.SemaphoreType.DMA,
            pltpu.SemaphoreType.DMA,
            pltpu.SemaphoreType.DMA,
            pltpu.SemaphoreType.DMA,
        ],
    )
    def k(hp_hbm, src_hbm, dst_hbm, zeros_hbm, out_hbm, src_v, dst_v,
          rows_a, rows_b, acc, gsem, hsem, isem0, isem1, initsem):
        c = lax.axis_index("c")
        s = lax.axis_index("s")
        base = (c * NS + s) * rows_per_tile

        @pl.when(c == 0)
        def _():
            pltpu.async_copy(hp_hbm.at[pl.ds(s * stripe, stripe)],
                             acc.at[pl.ds(s * stripe, stripe)], initsem)

        @pl.when(c != 0)
        def _():
            pltpu.async_copy(zeros_hbm.at[pl.ds(s * stripe, stripe)],
                             acc.at[pl.ds(s * stripe, stripe)], initsem)

        pltpu.async_copy(src_hbm.at[pl.ds(base, SB)], src_v.at[0], isem0)
        pltpu.async_copy(dst_hbm.at[pl.ds(base, SB)], dst_v.at[0], isem0)
        pltpu.make_async_copy(src_hbm.at[pl.ds(base, SB)], src_v.at[0],
                              isem0).wait()
        pltpu.make_async_copy(dst_hbm.at[pl.ds(base, SB)], dst_v.at[0],
                              isem0).wait()
        pltpu.async_copy(hp_hbm.at[src_v.at[0].at[0]], rows_a, gsem)
        pltpu.async_copy(hp_hbm.at[src_v.at[0].at[1]], rows_b, hsem)
        pltpu.make_async_copy(hp_hbm.at[pl.ds(s * stripe, stripe)],
                              acc.at[pl.ds(s * stripe, stripe)],
                              initsem).wait()
        plsc.subcore_barrier()

        @pl.loop(0, n_sb)
        def _(sb):
            p = sb % 2
            r_next = base + (sb + 1) * SB

            @pl.when(sb < n_sb - 1)
            def _():
                pltpu.async_copy(src_hbm.at[pl.ds(r_next, SB)],
                                 src_v.at[1 - p], isem1)
                pltpu.async_copy(dst_hbm.at[pl.ds(r_next, SB)],
                                 dst_v.at[1 - p], isem1)

            sidx = src_v.at[p]
            didx = dst_v.at[p]

            @pl.loop(0, SB // 2)
            def _(i):
                ca = 2 * i
                pltpu.make_async_copy(hp_hbm.at[sidx.at[ca]], rows_a,
                                      gsem).wait()
                pltpu.sync_copy(rows_a, acc.at[didx.at[ca]], add=True)

                @pl.when(i < SB // 2 - 1)
                def _():
                    pltpu.async_copy(hp_hbm.at[sidx.at[ca + 2]], rows_a, gsem)

                pltpu.make_async_copy(hp_hbm.at[sidx.at[ca + 1]], rows_b,
                                      hsem).wait()
                pltpu.sync_copy(rows_b, acc.at[didx.at[ca + 1]], add=True)

                @pl.when(i < SB // 2 - 1)
                def _():
                    pltpu.async_copy(hp_hbm.at[sidx.at[ca + 3]], rows_b, hsem)

            @pl.when(sb < n_sb - 1)
            def _():
                pltpu.make_async_copy(src_hbm.at[pl.ds(r_next, SB)],
                                      src_v.at[1 - p], isem1).wait()
                pltpu.make_async_copy(dst_hbm.at[pl.ds(r_next, SB)],
                                      dst_v.at[1 - p], isem1).wait()
                pltpu.async_copy(hp_hbm.at[src_v.at[1 - p].at[0]], rows_a,
                                 gsem)
                pltpu.async_copy(hp_hbm.at[src_v.at[1 - p].at[1]], rows_b,
                                 hsem)

        plsc.subcore_barrier()
        pltpu.sync_copy(acc.at[pl.ds(s * stripe, stripe)],
                        out_hbm.at[c].at[pl.ds(s * stripe, stripe)])

    return k(hp, src2d, dst2d, zeros2d)


def _dinv_col(hist_ref):
    return lax.rsqrt(1.0 + hist_ref[0] + hist_ref[1])


def _pad_rows(h):
    return jnp.concatenate(
        [h, jnp.zeros((NPAD - N, h.shape[1]), jnp.float32)], axis=0)


def _tc_layer1(x, W1, hist3):
    def body(x_ref, w_ref, hist_ref, out_ref):
        dinv = _dinv_col(hist_ref)
        h = jnp.dot(x_ref[...] * dinv[:N], w_ref[...],
                    preferred_element_type=jnp.float32)
        out_ref[...] = _pad_rows(h)

    return pl.pallas_call(
        body,
        out_shape=jax.ShapeDtypeStruct((NPAD, H), jnp.float32),
    )(x, W1, hist3)


def _tc_layer2(agg1, hist3, b1, W2):
    def body(agg_ref, hist_ref, b_ref, w_ref, out_ref):
        dinv = _dinv_col(hist_ref)
        aggc = agg_ref[0, :N] + agg_ref[1, :N]
        out1 = jax.nn.relu(aggc * dinv[:N] + b_ref[...])
        h2 = jnp.dot(out1, w_ref[...], preferred_element_type=jnp.float32)
        out_ref[...] = _pad_rows(h2 * dinv[:N])

    return pl.pallas_call(
        body,
        out_shape=jax.ShapeDtypeStruct((NPAD, H), jnp.float32),
    )(agg1, hist3, b1, W2)


def _tc_pool(agg2, hist3, b2, batch2d, W3, b3):
    def body(agg_ref, hist_ref, b2_ref, batch_ref, w3_ref, b3_ref, out_ref):
        dinv = _dinv_col(hist_ref)
        aggc = agg_ref[0, :N] + agg_ref[1, :N]
        out2 = jax.nn.relu(aggc * dinv[:N] + b2_ref[...])
        gids = lax.broadcasted_iota(jnp.int32, (G, 1), 0)
        P = (batch_ref[...] == gids).astype(jnp.float32)
        counts = jnp.sum(P, axis=1, keepdims=True)
        pooled = jnp.dot(P, out2, preferred_element_type=jnp.float32)
        pooled = pooled / jnp.maximum(counts, 1.0)
        out_ref[...] = (
            jnp.dot(pooled, w3_ref[...], preferred_element_type=jnp.float32)
            + b3_ref[...])

    return pl.pallas_call(
        body,
        out_shape=jax.ShapeDtypeStruct((G, OUT), jnp.float32),
    )(agg2, hist3, b2, batch2d, W3, b3)


def kernel(x, edge_index, batch, W1, b1, W2, b2, W3, b3):
    src = edge_index[0]
    dst = edge_index[1]
    pad_ids = N + (jnp.arange(EPAD - E, dtype=jnp.int32) % (NPAD - N))
    src2d = jnp.concatenate([src, pad_ids]).reshape(-1, CHUNK)
    dst2d = jnp.concatenate([dst, pad_ids]).reshape(-1, CHUNK)
    ones_c = jnp.ones((CHUNK,), jnp.float32)
    zeros_n = jnp.zeros((NPAD,), jnp.float32)
    zeros2d = jnp.zeros((NPAD, H), jnp.float32)

    hist = _sc_degree(dst2d, ones_c, zeros_n)
    hist3 = hist.reshape(NC, NPAD, 1)
    hp1 = _tc_layer1(x, W1, hist3)
    agg1 = _sc_scatter(hp1, src2d, dst2d, zeros2d)
    hp2 = _tc_layer2(agg1, hist3, b1.reshape(1, H), W2)
    agg2 = _sc_scatter(hp2, src2d, dst2d, zeros2d)
    return _tc_pool(agg2, hist3, b2.reshape(1, H), batch.reshape(1, N),
                    W3, b3.reshape(1, OUT))


# speedup vs baseline: 1.0107x; 1.0038x over previous
import functools

import jax
import jax.numpy as jnp
from jax import lax
from jax.experimental import pallas as pl
from jax.experimental.pallas import tpu as pltpu
from jax.experimental.pallas import tpu_sc as plsc

N = 10000
E = 320000
D = 128
H = 128
OUT = 64
G = 64

NC = 2
NS = 16
NPAD = 10240
CHUNK = 128
BLK = 2
EPAD = 327680

_vector_mesh = plsc.VectorSubcoreMesh(core_axis_name="c", subcore_axis_name="s")


def _sc_degree(dst2d, ones_c, zeros_n):
    rows = dst2d.shape[0]
    rows_per_w = rows // (NC * NS)
    n_blocks = rows_per_w // BLK
    stripe = NPAD // NS

    @functools.partial(
        pl.kernel,
        out_type=jax.ShapeDtypeStruct((NC, NPAD), jnp.float32),
        mesh=_vector_mesh,
        scratch_types=[
            pltpu.VMEM((rows // (NC * NS), CHUNK), jnp.int32),
            pltpu.VMEM((CHUNK,), jnp.float32),
            pltpu.VMEM_SHARED((NPAD,), jnp.float32),
            pltpu.SemaphoreType.DMA,
        ],
    )
    def k(dst_hbm, ones_hbm, zeros_hbm, out_hbm, idx_v, ones_v, acc, ssem):
        c = lax.axis_index("c")
        s = lax.axis_index("s")
        w = c * NS + s
        pltpu.sync_copy(zeros_hbm.at[pl.ds(s * stripe, stripe)],
                        acc.at[pl.ds(s * stripe, stripe)])
        pltpu.sync_copy(ones_hbm, ones_v)
        pltpu.sync_copy(dst_hbm.at[pl.ds(w * rows_per_w, rows_per_w)], idx_v)
        plsc.subcore_barrier()
        scatters = [
            pltpu.async_copy(ones_v, acc.at[idx_v.at[j]], ssem, add=True)
            for j in range(rows_per_w)
        ]
        for cp in scatters:
            cp.wait()
        plsc.subcore_barrier()
        pltpu.sync_copy(acc.at[pl.ds(s * stripe, stripe)],
                        out_hbm.at[c].at[pl.ds(s * stripe, stripe)])

    return k(dst2d, ones_c, zeros_n)


def _sc_scatter(hp, src2d, dst2d):
    rows = src2d.shape[0]
    rows_per_tile = rows // (NC * NS)
    SB = 16
    n_sb = rows_per_tile // SB
    stripe = NPAD // NS

    @functools.partial(
        pl.kernel,
        out_type=jax.ShapeDtypeStruct((NC, NPAD, H), jnp.float32),
        mesh=_vector_mesh,
        scratch_types=[
            pltpu.VMEM((2, SB, CHUNK), jnp.int32),
            pltpu.VMEM((2, SB, CHUNK), jnp.int32),
            pltpu.VMEM((CHUNK, H), jnp.float32),
            pltpu.VMEM((CHUNK, H), jnp.float32),
            pltpu.VMEM_SHARED((NPAD, H), jnp.float32),
            pltpu.SemaphoreType.DMA,
            pltpu.SemaphoreType.DMA,
            pltpu.SemaphoreType.DMA,
            pltpu.SemaphoreType.DMA,
            pltpu.SemaphoreType.DMA,
        ],
    )
    def k(hp_hbm, src_hbm, dst_hbm, out_hbm, src_v, dst_v,
          rows_a, rows_b, acc, gsem, hsem, isem0, isem1, initsem):
        c = lax.axis_index("c")
        s = lax.axis_index("s")
        base = (c * NS + s) * rows_per_tile

        zero16 = jnp.zeros((16,), jnp.float32)

        @pl.loop(0, CHUNK)
        def _(r):
            for cb in range(H // 16):
                rows_a[r, pl.ds(cb * 16, 16)] = zero16

        for kk in range(stripe // CHUNK):
            pltpu.async_copy(rows_a,
                             acc.at[pl.ds(s * stripe + kk * CHUNK, CHUNK)],
                             initsem)
        pltpu.async_copy(src_hbm.at[pl.ds(base, SB)], src_v.at[0], isem0)
        pltpu.async_copy(dst_hbm.at[pl.ds(base, SB)], dst_v.at[0], isem0)
        pltpu.make_async_copy(src_hbm.at[pl.ds(base, SB)], src_v.at[0],
                              isem0).wait()
        pltpu.make_async_copy(dst_hbm.at[pl.ds(base, SB)], dst_v.at[0],
                              isem0).wait()
        pltpu.async_copy(hp_hbm.at[src_v.at[0].at[1]], rows_b, hsem)
        for kk in range(stripe // CHUNK):
            pltpu.make_async_copy(
                rows_a, acc.at[pl.ds(s * stripe + kk * CHUNK, CHUNK)],
                initsem).wait()
        pltpu.async_copy(hp_hbm.at[src_v.at[0].at[0]], rows_a, gsem)
        plsc.subcore_barrier()

        @pl.loop(0, n_sb)
        def _(sb):
            p = sb % 2
            r_next = base + (sb + 1) * SB

            @pl.when(sb < n_sb - 1)
            def _():
                pltpu.async_copy(src_hbm.at[pl.ds(r_next, SB)],
                                 src_v.at[1 - p], isem1)
                pltpu.async_copy(dst_hbm.at[pl.ds(r_next, SB)],
                                 dst_v.at[1 - p], isem1)

            sidx = src_v.at[p]
            didx = dst_v.at[p]

            @pl.loop(0, SB // 2)
            def _(i):
                ca = 2 * i
                pltpu.make_async_copy(hp_hbm.at[sidx.at[ca]], rows_a,
                                      gsem).wait()
                pltpu.sync_copy(rows_a, acc.at[didx.at[ca]], add=True)

                @pl.when(i < SB // 2 - 1)
                def _():
                    pltpu.async_copy(hp_hbm.at[sidx.at[ca + 2]], rows_a, gsem)

                pltpu.make_async_copy(hp_hbm.at[sidx.at[ca + 1]], rows_b,
                                      hsem).wait()
                pltpu.sync_copy(rows_b, acc.at[didx.at[ca + 1]], add=True)

                @pl.when(i < SB // 2 - 1)
                def _():
                    pltpu.async_copy(hp_hbm.at[sidx.at[ca + 3]], rows_b, hsem)

            @pl.when(sb < n_sb - 1)
            def _():
                pltpu.make_async_copy(src_hbm.at[pl.ds(r_next, SB)],
                                      src_v.at[1 - p], isem1).wait()
                pltpu.make_async_copy(dst_hbm.at[pl.ds(r_next, SB)],
                                      dst_v.at[1 - p], isem1).wait()
                pltpu.async_copy(hp_hbm.at[src_v.at[1 - p].at[0]], rows_a,
                                 gsem)
                pltpu.async_copy(hp_hbm.at[src_v.at[1 - p].at[1]], rows_b,
                                 hsem)

        plsc.subcore_barrier()
        pltpu.sync_copy(acc.at[pl.ds(s * stripe, stripe)],
                        out_hbm.at[c].at[pl.ds(s * stripe, stripe)])

    return k(hp, src2d, dst2d)


def _dinv_col(hist_ref):
    return lax.rsqrt(1.0 + hist_ref[0] + hist_ref[1])


def _pad_rows(h):
    return jnp.concatenate(
        [h, jnp.zeros((NPAD - N, h.shape[1]), jnp.float32)], axis=0)


def _tc_layer1(x, W1, hist3):
    def body(x_ref, w_ref, hist_ref, out_ref):
        dinv = _dinv_col(hist_ref)
        h = jnp.dot(x_ref[...] * dinv[:N], w_ref[...],
                    preferred_element_type=jnp.float32)
        out_ref[...] = _pad_rows(h)

    return pl.pallas_call(
        body,
        out_shape=jax.ShapeDtypeStruct((NPAD, H), jnp.float32),
    )(x, W1, hist3)


def _tc_layer2(agg1, hp1, hist3, b1, W2):
    def body(agg_ref, hp_ref, hist_ref, b_ref, w_ref, out_ref):
        dinv = _dinv_col(hist_ref)
        aggc = agg_ref[0, :N] + agg_ref[1, :N] + hp_ref[:N]
        out1 = jax.nn.relu(aggc * dinv[:N] + b_ref[...])
        h2 = jnp.dot(out1, w_ref[...], preferred_element_type=jnp.float32)
        out_ref[...] = _pad_rows(h2 * dinv[:N])

    return pl.pallas_call(
        body,
        out_shape=jax.ShapeDtypeStruct((NPAD, H), jnp.float32),
    )(agg1, hp1, hist3, b1, W2)


def _tc_pool(agg2, hp2, hist3, b2, batch2d, W3, b3):
    def body(agg_ref, hp_ref, hist_ref, b2_ref, batch_ref, w3_ref, b3_ref,
             out_ref):
        dinv = _dinv_col(hist_ref)
        aggc = agg_ref[0, :N] + agg_ref[1, :N] + hp_ref[:N]
        out2 = jax.nn.relu(aggc * dinv[:N] + b2_ref[...])
        gids = lax.broadcasted_iota(jnp.int32, (G, 1), 0)
        P = (batch_ref[...] == gids).astype(jnp.float32)
        counts = jnp.sum(P, axis=1, keepdims=True)
        pooled = jnp.dot(P, out2, preferred_element_type=jnp.float32)
        pooled = pooled / jnp.maximum(counts, 1.0)
        out_ref[...] = (
            jnp.dot(pooled, w3_ref[...], preferred_element_type=jnp.float32)
            + b3_ref[...])

    return pl.pallas_call(
        body,
        out_shape=jax.ShapeDtypeStruct((G, OUT), jnp.float32),
    )(agg2, hp2, hist3, b2, batch2d, W3, b3)


def kernel(x, edge_index, batch, W1, b1, W2, b2, W3, b3):
    src = edge_index[0]
    dst = edge_index[1]
    pad_ids = N + (jnp.arange(EPAD - E, dtype=jnp.int32) % (NPAD - N))
    src2d = jnp.concatenate([src, pad_ids]).reshape(-1, CHUNK)
    dst2d = jnp.concatenate([dst, pad_ids]).reshape(-1, CHUNK)
    ones_c = jnp.ones((CHUNK,), jnp.float32)
    zeros_n = jnp.zeros((NPAD,), jnp.float32)

    hist = _sc_degree(dst2d, ones_c, zeros_n)
    hist3 = hist.reshape(NC, NPAD, 1)
    hp1 = _tc_layer1(x, W1, hist3)
    agg1 = _sc_scatter(hp1, src2d, dst2d)
    hp2 = _tc_layer2(agg1, hp1, hist3, b1.reshape(1, H), W2)
    agg2 = _sc_scatter(hp2, src2d, dst2d)
    return _tc_pool(agg2, hp2, hist3, b2.reshape(1, H), batch.reshape(1, N),
                    W3, b3.reshape(1, OUT))


# speedup vs baseline: 1.0117x; 1.0010x over previous
import functools

import jax
import jax.numpy as jnp
from jax import lax
from jax.experimental import pallas as pl
from jax.experimental.pallas import tpu as pltpu
from jax.experimental.pallas import tpu_sc as plsc

N = 10000
E = 320000
D = 128
H = 128
OUT = 64
G = 64

NC = 2
NS = 16
NPAD = 10240
CHUNK = 128
EPAD = 327680

_vector_mesh = plsc.VectorSubcoreMesh(core_axis_name="c", subcore_axis_name="s")


def _sc_degree(dst2d, ones_c, zeros_n):
    rows = dst2d.shape[0]
    rows_per_w = rows // (NC * NS)
    stripe = NPAD // NS

    @functools.partial(
        pl.kernel,
        out_type=jax.ShapeDtypeStruct((NC, NPAD), jnp.float32),
        mesh=_vector_mesh,
        scratch_types=[
            pltpu.VMEM((rows // (NC * NS), CHUNK), jnp.int32),
            pltpu.VMEM((CHUNK,), jnp.float32),
            pltpu.VMEM_SHARED((NPAD,), jnp.float32),
            pltpu.SemaphoreType.DMA,
        ],
    )
    def k(dst_hbm, ones_hbm, zeros_hbm, out_hbm, idx_v, ones_v, acc, ssem):
        c = lax.axis_index("c")
        s = lax.axis_index("s")
        w = c * NS + s
        pltpu.sync_copy(zeros_hbm.at[pl.ds(s * stripe, stripe)],
                        acc.at[pl.ds(s * stripe, stripe)])
        pltpu.sync_copy(ones_hbm, ones_v)
        pltpu.sync_copy(dst_hbm.at[pl.ds(w * rows_per_w, rows_per_w)], idx_v)
        plsc.subcore_barrier()
        scatters = [
            pltpu.async_copy(ones_v, acc.at[idx_v.at[j]], ssem, add=True)
            for j in range(rows_per_w)
        ]
        for cp in scatters:
            cp.wait()
        plsc.subcore_barrier()
        pltpu.sync_copy(acc.at[pl.ds(s * stripe, stripe)],
                        out_hbm.at[c].at[pl.ds(s * stripe, stripe)])

    return k(dst2d, ones_c, zeros_n)


def _sc_scatter(hp, src2d, dst2d):
    rows = src2d.shape[0]
    rows_per_tile = rows // (NC * NS)
    SB = 16
    n_sb = rows_per_tile // SB
    stripe = NPAD // NS

    @functools.partial(
        pl.kernel,
        out_type=jax.ShapeDtypeStruct((NC, NPAD, H), jnp.float32),
        mesh=_vector_mesh,
        scratch_types=[
            pltpu.VMEM((2, SB, CHUNK), jnp.int32),
            pltpu.VMEM((2, SB, CHUNK), jnp.int32),
            pltpu.VMEM((CHUNK, H), jnp.float32),
            pltpu.VMEM((CHUNK, H), jnp.float32),
            pltpu.VMEM_SHARED((NPAD, H), jnp.float32),
            pltpu.SemaphoreType.DMA,
            pltpu.SemaphoreType.DMA,
            pltpu.SemaphoreType.DMA,
            pltpu.SemaphoreType.DMA,
            pltpu.SemaphoreType.DMA,
        ],
    )
    def k(hp_hbm, src_hbm, dst_hbm, out_hbm, src_v, dst_v,
          rows_a, rows_b, acc, gsem, hsem, isem0, isem1, initsem):
        c = lax.axis_index("c")
        s = lax.axis_index("s")
        base = (c * NS + s) * rows_per_tile

        zero16 = jnp.zeros((16,), jnp.float32)

        @pl.loop(0, CHUNK)
        def _(r):
            for cb in range(H // 16):
                rows_a[r, pl.ds(cb * 16, 16)] = zero16

        for kk in range(stripe // CHUNK):
            pltpu.async_copy(rows_a,
                             acc.at[pl.ds(s * stripe + kk * CHUNK, CHUNK)],
                             initsem)
        pltpu.async_copy(src_hbm.at[pl.ds(base, SB)], src_v.at[0], isem0)
        pltpu.async_copy(dst_hbm.at[pl.ds(base, SB)], dst_v.at[0], isem0)
        pltpu.make_async_copy(src_hbm.at[pl.ds(base, SB)], src_v.at[0],
                              isem0).wait()
        pltpu.make_async_copy(dst_hbm.at[pl.ds(base, SB)], dst_v.at[0],
                              isem0).wait()
        pltpu.async_copy(hp_hbm.at[src_v.at[0].at[1]], rows_b, hsem)
        for kk in range(stripe // CHUNK):
            pltpu.make_async_copy(
                rows_a, acc.at[pl.ds(s * stripe + kk * CHUNK, CHUNK)],
                initsem).wait()
        pltpu.async_copy(hp_hbm.at[src_v.at[0].at[0]], rows_a, gsem)
        plsc.subcore_barrier()

        @pl.loop(0, n_sb)
        def _(sb):
            p = sb % 2
            r_next = base + (sb + 1) * SB

            @pl.when(sb < n_sb - 1)
            def _():
                pltpu.async_copy(src_hbm.at[pl.ds(r_next, SB)],
                                 src_v.at[1 - p], isem1)
                pltpu.async_copy(dst_hbm.at[pl.ds(r_next, SB)],
                                 dst_v.at[1 - p], isem1)

            sidx = src_v.at[p]
            didx = dst_v.at[p]

            @pl.loop(0, SB // 2)
            def _(i):
                ca = 2 * i
                pltpu.make_async_copy(hp_hbm.at[sidx.at[ca]], rows_a,
                                      gsem).wait()
                pltpu.sync_copy(rows_a, acc.at[didx.at[ca]], add=True)

                @pl.when(i < SB // 2 - 1)
                def _():
                    pltpu.async_copy(hp_hbm.at[sidx.at[ca + 2]], rows_a, gsem)

                pltpu.make_async_copy(hp_hbm.at[sidx.at[ca + 1]], rows_b,
                                      hsem).wait()
                pltpu.sync_copy(rows_b, acc.at[didx.at[ca + 1]], add=True)

                @pl.when(i < SB // 2 - 1)
                def _():
                    pltpu.async_copy(hp_hbm.at[sidx.at[ca + 3]], rows_b, hsem)

            @pl.when(sb < n_sb - 1)
            def _():
                pltpu.make_async_copy(src_hbm.at[pl.ds(r_next, SB)],
                                      src_v.at[1 - p], isem1).wait()
                pltpu.make_async_copy(dst_hbm.at[pl.ds(r_next, SB)],
                                      dst_v.at[1 - p], isem1).wait()
                pltpu.async_copy(hp_hbm.at[src_v.at[1 - p].at[0]], rows_a,
                                 gsem)
                pltpu.async_copy(hp_hbm.at[src_v.at[1 - p].at[1]], rows_b,
                                 hsem)

        plsc.subcore_barrier()
        pltpu.sync_copy(acc.at[pl.ds(s * stripe, stripe)],
                        out_hbm.at[c].at[pl.ds(s * stripe, stripe)])

    return k(hp, src2d, dst2d)


def _dinv_col(hist_ref):
    return lax.rsqrt(1.0 + hist_ref[0] + hist_ref[1])


def _pad_rows(h):
    return jnp.concatenate(
        [h, jnp.zeros((NPAD - N, h.shape[1]), jnp.float32)], axis=0)


def _tc_layer1(x, W1, hist3):
    def body(x_ref, w_ref, hist_ref, out_ref):
        dinv = _dinv_col(hist_ref)
        h = jnp.dot(x_ref[...] * dinv[:N], w_ref[...],
                    preferred_element_type=jnp.float32)
        out_ref[...] = _pad_rows(h)

    return pl.pallas_call(
        body,
        out_shape=jax.ShapeDtypeStruct((NPAD, H), jnp.float32),
    )(x, W1, hist3)


def _tc_layer2(agg1, hp1, hist3, b1, W2):
    def body(agg_ref, hp_ref, hist_ref, b_ref, w_ref, out_ref):
        dinv = _dinv_col(hist_ref)
        aggc = agg_ref[0, :N] + agg_ref[1, :N] + hp_ref[:N]
        out1 = jax.nn.relu(aggc * dinv[:N] + b_ref[...])
        h2 = jnp.dot(out1, w_ref[...], preferred_element_type=jnp.float32)
        out_ref[...] = _pad_rows(h2 * dinv[:N])

    return pl.pallas_call(
        body,
        out_shape=jax.ShapeDtypeStruct((NPAD, H), jnp.float32),
    )(agg1, hp1, hist3, b1, W2)


def _tc_pool(agg2, hp2, hist3, b2, batch2d, W3, b3):
    def body(agg_ref, hp_ref, hist_ref, b2_ref, batch_ref, w3_ref, b3_ref,
             out_ref):
        dinv = _dinv_col(hist_ref)
        aggc = agg_ref[0, :N] + agg_ref[1, :N] + hp_ref[:N]
        out2 = jax.nn.relu(aggc * dinv[:N] + b2_ref[...])
        gids = lax.broadcasted_iota(jnp.int32, (G, 1), 0)
        P = (batch_ref[...] == gids).astype(jnp.float32)
        counts = jnp.sum(P, axis=1, keepdims=True)
        pooled = jnp.dot(P, out2, preferred_element_type=jnp.float32)
        pooled = pooled / jnp.maximum(counts, 1.0)
        out_ref[...] = (
            jnp.dot(pooled, w3_ref[...], preferred_element_type=jnp.float32)
            + b3_ref[...])

    return pl.pallas_call(
        body,
        out_shape=jax.ShapeDtypeStruct((G, OUT), jnp.float32),
    )(agg2, hp2, hist3, b2, batch2d, W3, b3)


def kernel(x, edge_index, batch, W1, b1, W2, b2, W3, b3):
    src = edge_index[0]
    dst = edge_index[1]
    pad_ids = N + (jnp.arange(EPAD - E, dtype=jnp.int32) % (NPAD - N))
    src2d = jnp.concatenate([src, pad_ids]).reshape(-1, CHUNK)
    dst2d = jnp.concatenate([dst, pad_ids]).reshape(-1, CHUNK)
    ones_c = jnp.ones((CHUNK,), jnp.float32)
    zeros_n = jnp.zeros((NPAD,), jnp.float32)

    hist = _sc_degree(dst2d, ones_c, zeros_n)
    hist3 = hist.reshape(NC, NPAD, 1)
    hp1 = _tc_layer1(x, W1, hist3)
    agg1 = _sc_scatter(hp1, src2d, dst2d)
    hp2 = _tc_layer2(agg1, hp1, hist3, b1.reshape(1, H), W2)
    agg2 = _sc_scatter(hp2, src2d, dst2d)
    return _tc_pool(agg2, hp2, hist3, b2.reshape(1, H), batch.reshape(1, N),
                    W3, b3.reshape(1, OUT))
